# baseline (speedup 1.0000x reference)
import functools

import jax
import jax.numpy as jnp
from jax import lax
from jax.experimental import pallas as pl
from jax.experimental.pallas import tpu as pltpu
from jax.experimental.pallas import tpu_sc as plsc

VOCAB_SIZE = 100000
EMBED_DIM = 128
BATCH = 1024
SEQ = 200

_NC = 2
_NS = 16
_NW = _NC * _NS
_N_TOTAL = BATCH * SEQ
_N_PER_W = _N_TOTAL // _NW
_CHUNK = 128
_N_CHUNKS = _N_PER_W // _CHUNK


_NBUF = 5
_FIRE = 3


def _gather_body(idx_hbm, table_hbm, out_hbm, idx_v, *scratch):
    rows = scratch[:_NBUF]
    gsems = scratch[_NBUF:2 * _NBUF]
    wsems = scratch[2 * _NBUF:3 * _NBUF]

    wid = lax.axis_index("s") * _NC + lax.axis_index("c")
    base = wid * _N_PER_W
    pltpu.sync_copy(idx_hbm.at[pl.ds(base, _N_PER_W)], idx_v)

    def gather(j, b):
        return pltpu.make_async_copy(
            table_hbm.at[idx_v.at[pl.ds(j * _CHUNK, _CHUNK)]], rows[b], gsems[b]
        )

    def write(j, b):
        return pltpu.make_async_copy(
            rows[b], out_hbm.at[pl.ds(base + j * _CHUNK, _CHUNK)], wsems[b]
        )

    for j in range(_FIRE):
        gather(j, j % _NBUF).start()

    def group(g, carry):
        for b in range(_NBUF):
            j = g * _NBUF + b
            nb = (b + _FIRE) % _NBUF

            @pl.when(j >= _NBUF - _FIRE)
            def _():
                write(j - (_NBUF - _FIRE), nb).wait()

            @pl.when(j + _FIRE < _N_CHUNKS)
            def _():
                gather(j + _FIRE, nb).start()

            gather(j, b).wait()
            write(j, b).start()
        return carry

    lax.fori_loop(0, _N_CHUNKS // _NBUF, group, 0)

    for j in range(_N_CHUNKS - (_NBUF - _FIRE), _N_CHUNKS):
        write(j, j % _NBUF).wait()


@functools.lru_cache(maxsize=1)
def _build_gather():
    return functools.partial(
        pl.kernel,
        mesh=plsc.VectorSubcoreMesh(core_axis_name="c", subcore_axis_name="s"),
        out_type=jax.ShapeDtypeStruct((_N_TOTAL, EMBED_DIM), jnp.float32),
        scratch_types=(
            [pltpu.VMEM((_N_PER_W,), jnp.int32)]
            + [pltpu.VMEM((_CHUNK, EMBED_DIM), jnp.float32)] * _NBUF
            + [pltpu.SemaphoreType.DMA] * (2 * _NBUF)
        ),
    )(_gather_body)


def kernel(input_ids, embedding):
    ids_flat = jnp.reshape(input_ids.astype(jnp.int32), (_N_TOTAL,))
    out = _build_gather()(ids_flat, embedding)
    return jnp.reshape(out, (BATCH, SEQ, EMBED_DIM))


# speedup vs baseline: 1.0048x; 1.0048x over previous
import functools

import jax
import jax.numpy as jnp
from jax import lax
from jax.experimental import pallas as pl
from jax.experimental.pallas import tpu as pltpu
from jax.experimental.pallas import tpu_sc as plsc

VOCAB_SIZE = 100000
EMBED_DIM = 128
BATCH = 1024
SEQ = 200

_NC = 2
_NS = 16
_NW = _NC * _NS
_B_PER_W = BATCH // _NW
_NBUF = 4
_FIRE = 2


def _gather_body(idx_hbm, table_hbm, out_hbm, idx_v, *scratch):
    rows = scratch[:_NBUF]
    gsems = scratch[_NBUF:2 * _NBUF]
    wsems = scratch[2 * _NBUF:3 * _NBUF]

    wid = lax.axis_index("s") * _NC + lax.axis_index("c")
    base = wid * _B_PER_W
    pltpu.sync_copy(idx_hbm.at[pl.ds(base * SEQ, _B_PER_W * SEQ)], idx_v)

    def gather(j, b):
        return pltpu.make_async_copy(
            table_hbm.at[idx_v.at[pl.ds(j * SEQ, SEQ)]], rows[b], gsems[b]
        )

    def write(j, b):
        return pltpu.make_async_copy(rows[b], out_hbm.at[base + j], wsems[b])

    for j in range(_FIRE):
        gather(j, j % _NBUF).start()

    def group(g, carry):
        for b in range(_NBUF):
            j = g * _NBUF + b
            nb = (b + _FIRE) % _NBUF

            @pl.when(j >= _NBUF - _FIRE)
            def _():
                write(j - (_NBUF - _FIRE), nb).wait()

            @pl.when(j + _FIRE < _B_PER_W)
            def _():
                gather(j + _FIRE, nb).start()

            gather(j, b).wait()
            write(j, b).start()
        return carry

    lax.fori_loop(0, _B_PER_W // _NBUF, group, 0)

    for j in range(_B_PER_W - (_NBUF - _FIRE), _B_PER_W):
        write(j, j % _NBUF).wait()


@functools.lru_cache(maxsize=1)
def _build_gather():
    return functools.partial(
        pl.kernel,
        mesh=plsc.VectorSubcoreMesh(core_axis_name="c", subcore_axis_name="s"),
        out_type=jax.ShapeDtypeStruct((BATCH, SEQ, EMBED_DIM), jnp.float32),
        scratch_types=(
            [pltpu.VMEM((_B_PER_W * SEQ,), jnp.int32)]
            + [pltpu.VMEM((SEQ, EMBED_DIM), jnp.float32)] * _NBUF
            + [pltpu.SemaphoreType.DMA] * (2 * _NBUF)
        ),
    )(_gather_body)


def kernel(input_ids, embedding):
    ids_flat = jnp.reshape(input_ids.astype(jnp.int32), (BATCH * SEQ,))
    return _build_gather()(ids_flat, embedding)
